# baseline (speedup 1.0000x reference)
import functools

import jax
import jax.numpy as jnp
from jax import lax
from jax.experimental import pallas as pl
from jax.experimental.pallas import tpu as pltpu
from jax.experimental.pallas import tpu_sc as plsc

_NC = 2
_NS = 16
_K = 80
_ROWS = 624
_STG = 48
_K2 = 128
_TAIL = 32

def _sc_mesh():
    return plsc.VectorSubcoreMesh(core_axis_name="c", subcore_axis_name="s",
                                  num_cores=_NC, num_subcores=_NS)


def _deg_sc(dst, n_nodes):
    e = dst.shape[0]
    ept = e // _NS
    nchunk = ept // _K

    def body(dst_hbm, out_hbm, idx_v, ones_v, zb_v, hist_sh, sem):
        c = lax.axis_index("c")
        s = lax.axis_index("s")

        @pl.when(c == 0)
        def _core0():
            for j in range(_K // 16):
                ones_v[pl.ds(j * 16, 16)] = jnp.ones((16,), jnp.float32)
            for j in range(_ROWS // 16):
                zb_v[pl.ds(j * 16, 16)] = jnp.zeros((16,), jnp.float32)
            pltpu.sync_copy(zb_v, hist_sh.at[pl.ds(s * _ROWS, _ROWS)])

            @pl.when(s == _NS - 1)
            def _tail_z():
                pltpu.sync_copy(zb_v.at[pl.ds(0, 16)],
                                hist_sh.at[pl.ds(_NS * _ROWS, n_nodes - _NS * _ROWS)])

            plsc.subcore_barrier()

            def chunk(i, carry):
                base = s * ept + i * _K
                pltpu.sync_copy(dst_hbm.at[pl.ds(base, _K)], idx_v)
                pltpu.sync_copy(ones_v, hist_sh.at[idx_v], add=True)
                return carry

            lax.fori_loop(0, nchunk, chunk, 0)
            plsc.subcore_barrier()
            pltpu.sync_copy(hist_sh.at[pl.ds(s * _ROWS, _ROWS)], zb_v)
            pltpu.sync_copy(zb_v, out_hbm.at[pl.ds(s * _ROWS, _ROWS)])

            @pl.when(s == _NS - 1)
            def _tail_o():
                tail = n_nodes - _NS * _ROWS
                pltpu.sync_copy(hist_sh.at[pl.ds(_NS * _ROWS, tail)],
                                zb_v.at[pl.ds(0, tail)])
                pltpu.sync_copy(zb_v.at[pl.ds(0, tail)],
                                out_hbm.at[pl.ds(_NS * _ROWS, tail)])

    fn = pl.kernel(
        body,
        out_type=jax.ShapeDtypeStruct((n_nodes,), jnp.float32),
        mesh=_sc_mesh(),
        scratch_types=[
            pltpu.VMEM((_K,), jnp.int32),
            pltpu.VMEM((_K,), jnp.float32),
            pltpu.VMEM((_ROWS,), jnp.float32),
            pltpu.VMEM_SHARED((n_nodes,), jnp.float32),
            pltpu.SemaphoreType.DMA,
        ],
    )
    return fn(dst)


def _scatter_sc(zs, src, dst, t_steps, n_nodes, h_dim):
    e = src.shape[0]
    ept = e // _NS
    nch = ept // _K2
    tail_e = ept - nch * _K2
    npair = nch // 2
    tail = n_nodes - _NS * _ROWS

    def body(zs_hbm, src_hbm, dst_hbm, out_hbm,
             is0, is1, is2, is3, id0, id1, id2, id3, ra, rb,
             idxs_t, idxd_t, stg0, stg1, acc_sh,
             si0, si1, si2, si3, sg0, sg1, ss0, ss1, sw0, sw1):
        idxs_r = [is0, is1, is2, is3]
        idxd_r = [id0, id1, id2, id3]
        rows_r = [ra, rb]
        semi = [si0, si1, si2, si3]
        semg = [sg0, sg1]
        sems = [ss0, ss1]
        stg = [stg0, stg1]
        semw = [sw0, sw1]
        c = lax.axis_index("c")
        s = lax.axis_index("s")
        e0 = s * ept

        def idx_issue(i, r):
            base = e0 + i * _K2
            pltpu.async_copy(src_hbm.at[pl.ds(base, _K2)], idxs_r[r], semi[r])
            pltpu.async_copy(dst_hbm.at[pl.ds(base, _K2)], idxd_r[r], semi[r])

        def idx_wait(r):
            pltpu.make_async_copy(src_hbm.at[pl.ds(0, _K2)], idxs_r[r],
                                  semi[r]).wait()
            pltpu.make_async_copy(dst_hbm.at[pl.ds(0, _K2)], idxd_r[r],
                                  semi[r]).wait()

        def add_off(ib, n16, t_n):
            for j in range(n16):
                ib[pl.ds(j * 16, 16)] = ib[pl.ds(j * 16, 16)] + t_n

        for ti in range(t_steps // _NC):
            t = c + _NC * ti
            t_n = t * n_nodes

            def stage_io(write_out):
                npiece = _ROWS // _STG

                def piece(i, carry):
                    hb = t_n + s * _ROWS + i * _STG
                    sp = s * _ROWS + i * _STG
                    for b in range(2):
                        @pl.when(i % 2 == b)
                        def _one():
                            @pl.when(i >= 2)
                            def _drain():
                                if write_out:
                                    pltpu.make_async_copy(
                                        stg[b], out_hbm.at[pl.ds(hb, _STG)],
                                        semw[b]).wait()
                                else:
                                    pltpu.make_async_copy(
                                        stg[b], acc_sh.at[pl.ds(sp, _STG)],
                                        semw[b]).wait()
                            if write_out:
                                pltpu.sync_copy(acc_sh.at[pl.ds(sp, _STG)], stg[b])
                                pltpu.async_copy(stg[b],
                                                 out_hbm.at[pl.ds(hb, _STG)],
                                                 semw[b])
                            else:
                                pltpu.sync_copy(zs_hbm.at[pl.ds(hb, _STG)], stg[b])
                                pltpu.async_copy(stg[b],
                                                 acc_sh.at[pl.ds(sp, _STG)],
                                                 semw[b])
                    return carry
                lax.fori_loop(0, npiece, piece, 0)
                for b in range(2):
                    if write_out:
                        pltpu.make_async_copy(stg[b],
                                              out_hbm.at[pl.ds(t_n, _STG)],
                                              semw[b]).wait()
                    else:
                        pltpu.make_async_copy(stg[b],
                                              acc_sh.at[pl.ds(0, _STG)],
                                              semw[b]).wait()

                @pl.when(s == _NS - 1)
                def _tailc():
                    hb = t_n + _NS * _ROWS
                    sp = _NS * _ROWS
                    if write_out:
                        pltpu.sync_copy(acc_sh.at[pl.ds(sp, tail)],
                                        stg0.at[pl.ds(0, tail)])
                        pltpu.sync_copy(stg0.at[pl.ds(0, tail)],
                                        out_hbm.at[pl.ds(hb, tail)])
                    else:
                        pltpu.sync_copy(zs_hbm.at[pl.ds(hb, tail)],
                                        stg0.at[pl.ds(0, tail)])
                        pltpu.sync_copy(stg0.at[pl.ds(0, tail)],
                                        acc_sh.at[pl.ds(sp, tail)])

            stage_io(write_out=False)
            plsc.subcore_barrier()

            idx_issue(0, 0)
            idx_issue(1, 1)
            idx_wait(0)
            add_off(idxs_r[0], _K2 // 16, t_n)
            pltpu.async_copy(zs_hbm.at[idxs_r[0]], rows_r[0], semg[0])

            def quad(q, carry):
                for u in range(4):
                    i = 4 * q + u
                    r0, r1, r2 = u % 4, (u + 1) % 4, (u + 2) % 4
                    b, nb = u % 2, (u + 1) % 2

                    @pl.when(i > 0)
                    def _wait_prev_scatter():
                        pltpu.make_async_copy(
                            rows_r[nb], acc_sh.at[idxd_r[(u + 3) % 4]],
                            sems[nb]).wait()

                    @pl.when(i + 1 < nch)
                    def _issue_next_gather():
                        idx_wait(r1)
                        add_off(idxs_r[r1], _K2 // 16, t_n)
                        pltpu.async_copy(zs_hbm.at[idxs_r[r1]], rows_r[nb],
                                         semg[nb])

                    pltpu.make_async_copy(zs_hbm.at[idxs_r[r0]], rows_r[b],
                                          semg[b]).wait()

                    @pl.when(i + 2 < nch)
                    def _prefetch_idx():
                        idx_issue(i + 2, r2)

                    pltpu.async_copy(rows_r[b], acc_sh.at[idxd_r[r0]],
                                     sems[b], add=True)
                return carry

            lax.fori_loop(0, nch // 4, quad, 0)
            pltpu.make_async_copy(rows_r[(nch - 1) % 2],
                                  acc_sh.at[idxd_r[(nch - 1) % 4]],
                                  sems[(nch - 1) % 2]).wait()

            tb = e0 + nch * _K2
            pltpu.sync_copy(src_hbm.at[pl.ds(tb, tail_e)], idxs_t)
            pltpu.sync_copy(dst_hbm.at[pl.ds(tb, tail_e)], idxd_t)
            add_off(idxs_t, tail_e // 16, t_n)
            pltpu.async_copy(zs_hbm.at[idxs_t],
                             rows_r[0].at[pl.ds(0, _TAIL)], semg[0]).wait()
            pltpu.sync_copy(rows_r[0].at[pl.ds(0, _TAIL)],
                            acc_sh.at[idxd_t], add=True)

            plsc.subcore_barrier()
            stage_io(write_out=True)

    fn = pl.kernel(
        body,
        out_type=jax.ShapeDtypeStruct((t_steps * n_nodes, h_dim), jnp.float32),
        mesh=_sc_mesh(),
        scratch_types=(
            [pltpu.VMEM((_K2,), jnp.int32)] * 8
            + [pltpu.VMEM((_K2, h_dim), jnp.float32)] * 2
            + [pltpu.VMEM((_TAIL,), jnp.int32)] * 2
            + [pltpu.VMEM((_STG, h_dim), jnp.float32)] * 2
            + [pltpu.VMEM_SHARED((n_nodes, h_dim), jnp.float32)]
            + [pltpu.SemaphoreType.DMA] * 10
        ),
    )
    return fn(zs, src, dst)


def _zs1_tc(x, w1, deg, nb):
    t_steps, n_nodes, f_dim = x.shape
    h_dim = w1.shape[1]

    def body(x_ref, w_ref, deg_ref, o_ref):
        dinv = lax.rsqrt(deg_ref[...] + 1.0)
        o_ref[0] = jnp.dot(x_ref[0].astype(jnp.bfloat16), w_ref[...],
                           preferred_element_type=jnp.float32) * dinv

    return pl.pallas_call(
        body,
        grid=(t_steps, n_nodes // nb),
        in_specs=[
            pl.BlockSpec((1, nb, f_dim), lambda t, i: (t, i, 0)),
            pl.BlockSpec((f_dim, h_dim), lambda t, i: (0, 0)),
            pl.BlockSpec((nb, 1), lambda t, i: (i, 0)),
        ],
        out_specs=pl.BlockSpec((1, nb, h_dim), lambda t, i: (t, i, 0)),
        out_shape=jax.ShapeDtypeStruct((t_steps, n_nodes, h_dim), jnp.float32),
    )(x, w1, deg)


def _zs2_tc(s1, w2, b1, deg, nb):
    t_steps, n_nodes, h_dim = s1.shape

    def body(s1_ref, w_ref, b_ref, deg_ref, o_ref):
        dinv = lax.rsqrt(deg_ref[...] + 1.0)
        y = jnp.maximum(s1_ref[0] * dinv + b_ref[...], 0.0)
        o_ref[0] = jnp.dot(y.astype(jnp.bfloat16), w_ref[...],
                           preferred_element_type=jnp.float32) * dinv

    return pl.pallas_call(
        body,
        grid=(t_steps, n_nodes // nb),
        in_specs=[
            pl.BlockSpec((1, nb, h_dim), lambda t, i: (t, i, 0)),
            pl.BlockSpec((h_dim, h_dim), lambda t, i: (0, 0)),
            pl.BlockSpec((1, h_dim), lambda t, i: (0, 0)),
            pl.BlockSpec((nb, 1), lambda t, i: (i, 0)),
        ],
        out_specs=pl.BlockSpec((1, nb, h_dim), lambda t, i: (t, i, 0)),
        out_shape=jax.ShapeDtypeStruct((t_steps, n_nodes, h_dim), jnp.float32),
    )(s1, w2, b1, deg)


def _lstm_tc(s2, deg, b2, wih0t, whh0t, bs0, wih1t, whh1t, bs1, wfct, bfc, nb):
    t_steps, n_nodes, h_dim = s2.shape
    out_dim = wfct.shape[1]

    def body(s2_ref, deg_ref, b2_ref, wih0_ref, whh0_ref, bs0_ref,
             wih1_ref, whh1_ref, bs1_ref, wfc_ref, bfc_ref, o_ref):
        dinv = lax.rsqrt(deg_ref[...] + 1.0)

        def cell(xt, h, c, wih, whh, bs):
            g = (jnp.dot(xt.astype(jnp.bfloat16), wih[...],
                         preferred_element_type=jnp.float32)
                 + jnp.dot(h.astype(jnp.bfloat16), whh[...],
                           preferred_element_type=jnp.float32)
                 + bs[...])
            i = jax.nn.sigmoid(g[:, :h_dim])
            f = jax.nn.sigmoid(g[:, h_dim:2 * h_dim])
            gg = jnp.tanh(g[:, 2 * h_dim:3 * h_dim])
            o = jax.nn.sigmoid(g[:, 3 * h_dim:])
            c = f * c + i * gg
            h = o * jnp.tanh(c)
            return h, c

        h0 = jnp.zeros((nb, h_dim), jnp.float32)
        c0 = jnp.zeros((nb, h_dim), jnp.float32)
        hs = []
        for t in range(t_steps):
            xt = jnp.maximum(s2_ref[t] * dinv + b2_ref[...], 0.0)
            h0, c0 = cell(xt, h0, c0, wih0_ref, whh0_ref, bs0_ref)
            hs.append(h0)
        h1 = jnp.zeros((nb, h_dim), jnp.float32)
        c1 = jnp.zeros((nb, h_dim), jnp.float32)
        for t in range(t_steps):
            h1, c1 = cell(hs[t], h1, c1, wih1_ref, whh1_ref, bs1_ref)
        o_ref[...] = (jnp.dot(h1.astype(jnp.bfloat16), wfc_ref[...],
                              preferred_element_type=jnp.float32)
                      + bfc_ref[...])

    full = lambda *shape: pl.BlockSpec(shape, lambda i: tuple(0 for _ in shape))
    return pl.pallas_call(
        body,
        grid=(n_nodes // nb,),
        in_specs=[
            pl.BlockSpec((t_steps, nb, h_dim), lambda i: (0, i, 0)),
            pl.BlockSpec((nb, 1), lambda i: (i, 0)),
            full(1, h_dim),
            full(h_dim, 4 * h_dim), full(h_dim, 4 * h_dim), full(1, 4 * h_dim),
            full(h_dim, 4 * h_dim), full(h_dim, 4 * h_dim), full(1, 4 * h_dim),
            full(h_dim, out_dim), full(1, out_dim),
        ],
        out_specs=pl.BlockSpec((nb, out_dim), lambda i: (i, 0)),
        out_shape=jax.ShapeDtypeStruct((n_nodes, out_dim), jnp.float32),
    )(s2, deg, b2, wih0t, whh0t, bs0, wih1t, whh1t, bs1, wfct, bfc)


def kernel(x, edge_index, W1, b1, W2, b2, Wih0, Whh0, bih0, bhh0,
           Wih1, Whh1, bih1, bhh1, Wfc, bfc):
    bsz, t_steps, n_nodes, f_dim = x.shape
    h_dim = W1.shape[1]
    src, dst = edge_index[0], edge_index[1]

    deg = _deg_sc(dst, n_nodes).reshape(n_nodes, 1)

    nb_mm = 2000 if n_nodes % 2000 == 0 else n_nodes
    nb_ls = 1000 if n_nodes % 1000 == 0 else n_nodes
    bf = jnp.bfloat16
    xs3 = x.reshape(t_steps, n_nodes, f_dim)
    zs1 = _zs1_tc(xs3, W1.astype(bf), deg, nb=nb_mm)
    s1 = _scatter_sc(zs1.reshape(t_steps * n_nodes, h_dim), src, dst,
                     t_steps, n_nodes, h_dim).reshape(t_steps, n_nodes, h_dim)
    zs2 = _zs2_tc(s1, W2.astype(bf), b1.reshape(1, h_dim), deg, nb=nb_mm)
    s2 = _scatter_sc(zs2.reshape(t_steps * n_nodes, h_dim), src, dst,
                     t_steps, n_nodes, h_dim).reshape(t_steps, n_nodes, h_dim)

    out = _lstm_tc(
        s2, deg, b2.reshape(1, h_dim),
        Wih0.T.astype(bf), Whh0.T.astype(bf),
        (bih0 + bhh0).reshape(1, 4 * h_dim),
        Wih1.T.astype(bf), Whh1.T.astype(bf),
        (bih1 + bhh1).reshape(1, 4 * h_dim),
        Wfc.T.astype(bf), bfc.reshape(1, -1), nb=nb_ls)
    return out.reshape(bsz, n_nodes, -1)


# speedup vs baseline: 1.0115x; 1.0115x over previous
import functools

import jax
import jax.numpy as jnp
from jax import lax
from jax.experimental import pallas as pl
from jax.experimental.pallas import tpu as pltpu
from jax.experimental.pallas import tpu_sc as plsc

_NC = 2
_NS = 16
_K = 80
_ROWS = 624
_STG = 48
_K2 = 128
_TAIL = 32

def _sc_mesh():
    return plsc.VectorSubcoreMesh(core_axis_name="c", subcore_axis_name="s",
                                  num_cores=_NC, num_subcores=_NS)


def _deg_sc(dst, n_nodes):
    e = dst.shape[0]
    ept = e // (_NC * _NS)
    nchunk = ept // _K

    def body(dst_hbm, out_hbm, idx_v, ones_v, zb_v, hist_sh, sem):
        c = lax.axis_index("c")
        s = lax.axis_index("s")
        for j in range(_K // 16):
            ones_v[pl.ds(j * 16, 16)] = jnp.ones((16,), jnp.float32)
        for j in range(_ROWS // 16):
            zb_v[pl.ds(j * 16, 16)] = jnp.zeros((16,), jnp.float32)
        pltpu.sync_copy(zb_v, hist_sh.at[pl.ds(s * _ROWS, _ROWS)])

        @pl.when(s == _NS - 1)
        def _tail_z():
            pltpu.sync_copy(zb_v.at[pl.ds(0, 16)],
                            hist_sh.at[pl.ds(_NS * _ROWS, n_nodes - _NS * _ROWS)])

        plsc.subcore_barrier()

        def chunk(i, carry):
            base = (c * _NS + s) * ept + i * _K
            pltpu.sync_copy(dst_hbm.at[pl.ds(base, _K)], idx_v)
            pltpu.sync_copy(ones_v, hist_sh.at[idx_v], add=True)
            return carry

        lax.fori_loop(0, nchunk, chunk, 0)
        plsc.subcore_barrier()
        o0 = c * n_nodes
        pltpu.sync_copy(hist_sh.at[pl.ds(s * _ROWS, _ROWS)], zb_v)
        pltpu.sync_copy(zb_v, out_hbm.at[pl.ds(o0 + s * _ROWS, _ROWS)])

        @pl.when(s == _NS - 1)
        def _tail_o():
            tail = n_nodes - _NS * _ROWS
            pltpu.sync_copy(hist_sh.at[pl.ds(_NS * _ROWS, tail)],
                            zb_v.at[pl.ds(0, tail)])
            pltpu.sync_copy(zb_v.at[pl.ds(0, tail)],
                            out_hbm.at[pl.ds(o0 + _NS * _ROWS, tail)])

    fn = pl.kernel(
        body,
        out_type=jax.ShapeDtypeStruct((_NC * n_nodes,), jnp.float32),
        mesh=_sc_mesh(),
        scratch_types=[
            pltpu.VMEM((_K,), jnp.int32),
            pltpu.VMEM((_K,), jnp.float32),
            pltpu.VMEM((_ROWS,), jnp.float32),
            pltpu.VMEM_SHARED((n_nodes,), jnp.float32),
            pltpu.SemaphoreType.DMA,
        ],
    )
    return fn(dst)


def _scatter_sc(zs, src, dst, t_steps, n_nodes, h_dim):
    e = src.shape[0]
    ept = e // _NS
    nch = ept // _K2
    tail_e = ept - nch * _K2
    npair = nch // 2
    tail = n_nodes - _NS * _ROWS

    def body(zs_hbm, src_hbm, dst_hbm, out_hbm,
             is0, is1, is2, is3, id0, id1, id2, id3, ra, rb,
             idxs_t, idxd_t, stg0, stg1, acc_sh,
             si0, si1, si2, si3, sg0, sg1, ss0, ss1, sw0, sw1):
        idxs_r = [is0, is1, is2, is3]
        idxd_r = [id0, id1, id2, id3]
        rows_r = [ra, rb]
        semi = [si0, si1, si2, si3]
        semg = [sg0, sg1]
        sems = [ss0, ss1]
        stg = [stg0, stg1]
        semw = [sw0, sw1]
        c = lax.axis_index("c")
        s = lax.axis_index("s")
        e0 = s * ept

        def idx_issue(i, r):
            base = e0 + i * _K2
            pltpu.async_copy(src_hbm.at[pl.ds(base, _K2)], idxs_r[r], semi[r])
            pltpu.async_copy(dst_hbm.at[pl.ds(base, _K2)], idxd_r[r], semi[r])

        def idx_wait(r):
            pltpu.make_async_copy(src_hbm.at[pl.ds(0, _K2)], idxs_r[r],
                                  semi[r]).wait()
            pltpu.make_async_copy(dst_hbm.at[pl.ds(0, _K2)], idxd_r[r],
                                  semi[r]).wait()

        def add_off(ib, n16, t_n):
            for j in range(n16):
                ib[pl.ds(j * 16, 16)] = ib[pl.ds(j * 16, 16)] + t_n

        for ti in range(t_steps // _NC):
            t = c + _NC * ti
            t_n = t * n_nodes

            def stage_io(write_out):
                npiece = _ROWS // _STG

                def piece(i, carry):
                    hb = t_n + s * _ROWS + i * _STG
                    sp = s * _ROWS + i * _STG
                    for b in range(2):
                        @pl.when(i % 2 == b)
                        def _one():
                            @pl.when(i >= 2)
                            def _drain():
                                if write_out:
                                    pltpu.make_async_copy(
                                        stg[b], out_hbm.at[pl.ds(hb, _STG)],
                                        semw[b]).wait()
                                else:
                                    pltpu.make_async_copy(
                                        stg[b], acc_sh.at[pl.ds(sp, _STG)],
                                        semw[b]).wait()
                            if write_out:
                                pltpu.sync_copy(acc_sh.at[pl.ds(sp, _STG)], stg[b])
                                pltpu.async_copy(stg[b],
                                                 out_hbm.at[pl.ds(hb, _STG)],
                                                 semw[b])
                            else:
                                pltpu.sync_copy(zs_hbm.at[pl.ds(hb, _STG)], stg[b])
                                pltpu.async_copy(stg[b],
                                                 acc_sh.at[pl.ds(sp, _STG)],
                                                 semw[b])
                    return carry
                lax.fori_loop(0, npiece, piece, 0)
                for b in range(2):
                    if write_out:
                        pltpu.make_async_copy(stg[b],
                                              out_hbm.at[pl.ds(t_n, _STG)],
                                              semw[b]).wait()
                    else:
                        pltpu.make_async_copy(stg[b],
                                              acc_sh.at[pl.ds(0, _STG)],
                                              semw[b]).wait()

                @pl.when(s == _NS - 1)
                def _tailc():
                    hb = t_n + _NS * _ROWS
                    sp = _NS * _ROWS
                    if write_out:
                        pltpu.sync_copy(acc_sh.at[pl.ds(sp, tail)],
                                        stg0.at[pl.ds(0, tail)])
                        pltpu.sync_copy(stg0.at[pl.ds(0, tail)],
                                        out_hbm.at[pl.ds(hb, tail)])
                    else:
                        pltpu.sync_copy(zs_hbm.at[pl.ds(hb, tail)],
                                        stg0.at[pl.ds(0, tail)])
                        pltpu.sync_copy(stg0.at[pl.ds(0, tail)],
                                        acc_sh.at[pl.ds(sp, tail)])

            stage_io(write_out=False)
            plsc.subcore_barrier()

            idx_issue(0, 0)
            idx_issue(1, 1)
            idx_wait(0)
            add_off(idxs_r[0], _K2 // 16, t_n)
            pltpu.async_copy(zs_hbm.at[idxs_r[0]], rows_r[0], semg[0])

            def quad(q, carry):
                for u in range(4):
                    i = 4 * q + u
                    r0, r1, r2 = u % 4, (u + 1) % 4, (u + 2) % 4
                    b, nb = u % 2, (u + 1) % 2

                    @pl.when(i > 0)
                    def _wait_prev_scatter():
                        pltpu.make_async_copy(
                            rows_r[nb], acc_sh.at[idxd_r[(u + 3) % 4]],
                            sems[nb]).wait()

                    @pl.when(i + 1 < nch)
                    def _issue_next_gather():
                        idx_wait(r1)
                        add_off(idxs_r[r1], _K2 // 16, t_n)
                        pltpu.async_copy(zs_hbm.at[idxs_r[r1]], rows_r[nb],
                                         semg[nb])

                    pltpu.make_async_copy(zs_hbm.at[idxs_r[r0]], rows_r[b],
                                          semg[b]).wait()

                    @pl.when(i + 2 < nch)
                    def _prefetch_idx():
                        idx_issue(i + 2, r2)

                    pltpu.async_copy(rows_r[b], acc_sh.at[idxd_r[r0]],
                                     sems[b], add=True)
                return carry

            lax.fori_loop(0, nch // 4, quad, 0)
            pltpu.make_async_copy(rows_r[(nch - 1) % 2],
                                  acc_sh.at[idxd_r[(nch - 1) % 4]],
                                  sems[(nch - 1) % 2]).wait()

            tb = e0 + nch * _K2
            pltpu.sync_copy(src_hbm.at[pl.ds(tb, tail_e)], idxs_t)
            pltpu.sync_copy(dst_hbm.at[pl.ds(tb, tail_e)], idxd_t)
            add_off(idxs_t, tail_e // 16, t_n)
            pltpu.async_copy(zs_hbm.at[idxs_t],
                             rows_r[0].at[pl.ds(0, _TAIL)], semg[0]).wait()
            pltpu.sync_copy(rows_r[0].at[pl.ds(0, _TAIL)],
                            acc_sh.at[idxd_t], add=True)

            plsc.subcore_barrier()
            stage_io(write_out=True)

    fn = pl.kernel(
        body,
        out_type=jax.ShapeDtypeStruct((t_steps * n_nodes, h_dim), jnp.float32),
        mesh=_sc_mesh(),
        scratch_types=(
            [pltpu.VMEM((_K2,), jnp.int32)] * 8
            + [pltpu.VMEM((_K2, h_dim), jnp.float32)] * 2
            + [pltpu.VMEM((_TAIL,), jnp.int32)] * 2
            + [pltpu.VMEM((_STG, h_dim), jnp.float32)] * 2
            + [pltpu.VMEM_SHARED((n_nodes, h_dim), jnp.float32)]
            + [pltpu.SemaphoreType.DMA] * 10
        ),
    )
    return fn(zs, src, dst)


def _zs1_tc(x, w1, deg, nb):
    t_steps, n_nodes, f_dim = x.shape
    h_dim = w1.shape[1]

    def body(x_ref, w_ref, deg_ref, o_ref):
        dinv = lax.rsqrt(deg_ref[0] + deg_ref[1] + 1.0)
        o_ref[0] = jnp.dot(x_ref[0].astype(jnp.bfloat16), w_ref[...],
                           preferred_element_type=jnp.float32) * dinv

    return pl.pallas_call(
        body,
        grid=(t_steps, n_nodes // nb),
        in_specs=[
            pl.BlockSpec((1, nb, f_dim), lambda t, i: (t, i, 0)),
            pl.BlockSpec((f_dim, h_dim), lambda t, i: (0, 0)),
            pl.BlockSpec((2, nb, 1), lambda t, i: (0, i, 0)),
        ],
        out_specs=pl.BlockSpec((1, nb, h_dim), lambda t, i: (t, i, 0)),
        out_shape=jax.ShapeDtypeStruct((t_steps, n_nodes, h_dim), jnp.float32),
    )(x, w1, deg)


def _zs2_tc(s1, w2, b1, deg, nb):
    t_steps, n_nodes, h_dim = s1.shape

    def body(s1_ref, w_ref, b_ref, deg_ref, o_ref):
        dinv = lax.rsqrt(deg_ref[0] + deg_ref[1] + 1.0)
        y = jnp.maximum(s1_ref[0] * dinv + b_ref[...], 0.0)
        o_ref[0] = jnp.dot(y.astype(jnp.bfloat16), w_ref[...],
                           preferred_element_type=jnp.float32) * dinv

    return pl.pallas_call(
        body,
        grid=(t_steps, n_nodes // nb),
        in_specs=[
            pl.BlockSpec((1, nb, h_dim), lambda t, i: (t, i, 0)),
            pl.BlockSpec((h_dim, h_dim), lambda t, i: (0, 0)),
            pl.BlockSpec((1, h_dim), lambda t, i: (0, 0)),
            pl.BlockSpec((2, nb, 1), lambda t, i: (0, i, 0)),
        ],
        out_specs=pl.BlockSpec((1, nb, h_dim), lambda t, i: (t, i, 0)),
        out_shape=jax.ShapeDtypeStruct((t_steps, n_nodes, h_dim), jnp.float32),
    )(s1, w2, b1, deg)


def _lstm_tc(s2, deg, b2, wih0t, whh0t, bs0, wih1t, whh1t, bs1, wfct, bfc, nb):
    t_steps, n_nodes, h_dim = s2.shape
    out_dim = wfct.shape[1]

    def body(s2_ref, deg_ref, b2_ref, wih0_ref, whh0_ref, bs0_ref,
             wih1_ref, whh1_ref, bs1_ref, wfc_ref, bfc_ref, o_ref):
        dinv = lax.rsqrt(deg_ref[0] + deg_ref[1] + 1.0)

        def cell(xt, h, c, wih, whh, bs):
            g = (jnp.dot(xt.astype(jnp.bfloat16), wih[...],
                         preferred_element_type=jnp.float32)
                 + jnp.dot(h.astype(jnp.bfloat16), whh[...],
                           preferred_element_type=jnp.float32)
                 + bs[...])
            i = jax.nn.sigmoid(g[:, :h_dim])
            f = jax.nn.sigmoid(g[:, h_dim:2 * h_dim])
            gg = jnp.tanh(g[:, 2 * h_dim:3 * h_dim])
            o = jax.nn.sigmoid(g[:, 3 * h_dim:])
            c = f * c + i * gg
            h = o * jnp.tanh(c)
            return h, c

        h0 = jnp.zeros((nb, h_dim), jnp.float32)
        c0 = jnp.zeros((nb, h_dim), jnp.float32)
        hs = []
        for t in range(t_steps):
            xt = jnp.maximum(s2_ref[t] * dinv + b2_ref[...], 0.0)
            h0, c0 = cell(xt, h0, c0, wih0_ref, whh0_ref, bs0_ref)
            hs.append(h0)
        h1 = jnp.zeros((nb, h_dim), jnp.float32)
        c1 = jnp.zeros((nb, h_dim), jnp.float32)
        for t in range(t_steps):
            h1, c1 = cell(hs[t], h1, c1, wih1_ref, whh1_ref, bs1_ref)
        o_ref[...] = (jnp.dot(h1.astype(jnp.bfloat16), wfc_ref[...],
                              preferred_element_type=jnp.float32)
                      + bfc_ref[...])

    full = lambda *shape: pl.BlockSpec(shape, lambda i: tuple(0 for _ in shape))
    return pl.pallas_call(
        body,
        grid=(n_nodes // nb,),
        in_specs=[
            pl.BlockSpec((t_steps, nb, h_dim), lambda i: (0, i, 0)),
            pl.BlockSpec((2, nb, 1), lambda i: (0, i, 0)),
            full(1, h_dim),
            full(h_dim, 4 * h_dim), full(h_dim, 4 * h_dim), full(1, 4 * h_dim),
            full(h_dim, 4 * h_dim), full(h_dim, 4 * h_dim), full(1, 4 * h_dim),
            full(h_dim, out_dim), full(1, out_dim),
        ],
        out_specs=pl.BlockSpec((nb, out_dim), lambda i: (i, 0)),
        out_shape=jax.ShapeDtypeStruct((n_nodes, out_dim), jnp.float32),
    )(s2, deg, b2, wih0t, whh0t, bs0, wih1t, whh1t, bs1, wfct, bfc)


def kernel(x, edge_index, W1, b1, W2, b2, Wih0, Whh0, bih0, bhh0,
           Wih1, Whh1, bih1, bhh1, Wfc, bfc):
    bsz, t_steps, n_nodes, f_dim = x.shape
    h_dim = W1.shape[1]
    src, dst = edge_index[0], edge_index[1]

    deg = _deg_sc(dst, n_nodes).reshape(2, n_nodes, 1)

    nb_mm = 2000 if n_nodes % 2000 == 0 else n_nodes
    nb_ls = 1000 if n_nodes % 1000 == 0 else n_nodes
    bf = jnp.bfloat16
    xs3 = x.reshape(t_steps, n_nodes, f_dim)
    zs1 = _zs1_tc(xs3, W1.astype(bf), deg, nb=nb_mm)
    s1 = _scatter_sc(zs1.reshape(t_steps * n_nodes, h_dim), src, dst,
                     t_steps, n_nodes, h_dim).reshape(t_steps, n_nodes, h_dim)
    zs2 = _zs2_tc(s1, W2.astype(bf), b1.reshape(1, h_dim), deg, nb=nb_mm)
    s2 = _scatter_sc(zs2.reshape(t_steps * n_nodes, h_dim), src, dst,
                     t_steps, n_nodes, h_dim).reshape(t_steps, n_nodes, h_dim)

    out = _lstm_tc(
        s2, deg, b2.reshape(1, h_dim),
        Wih0.T.astype(bf), Whh0.T.astype(bf),
        (bih0 + bhh0).reshape(1, 4 * h_dim),
        Wih1.T.astype(bf), Whh1.T.astype(bf),
        (bih1 + bhh1).reshape(1, 4 * h_dim),
        Wfc.T.astype(bf), bfc.reshape(1, -1), nb=nb_ls)
    return out.reshape(bsz, n_nodes, -1)


# speedup vs baseline: 1.0131x; 1.0016x over previous
import jax
import jax.numpy as jnp
from jax import lax
from jax.experimental import pallas as pl
from jax.experimental.pallas import tpu as pltpu
from jax.experimental.pallas import tpu_sc as plsc

_NC = 2
_NS = 16
_K = 80
_ROWS = 624
_STG = 48
_K2 = 128
_TAIL = 32

def _sc_mesh():
    return plsc.VectorSubcoreMesh(core_axis_name="c", subcore_axis_name="s",
                                  num_cores=_NC, num_subcores=_NS)


def _deg_sc(dst, n_nodes):
    e = dst.shape[0]
    ept = e // (_NC * _NS)
    nchunk = ept // _K

    def body(dst_hbm, out_hbm, idx_v, ones_v, zb_v, hist_sh, sem):
        c = lax.axis_index("c")
        s = lax.axis_index("s")
        for j in range(_K // 16):
            ones_v[pl.ds(j * 16, 16)] = jnp.ones((16,), jnp.float32)
        for j in range(_ROWS // 16):
            zb_v[pl.ds(j * 16, 16)] = jnp.zeros((16,), jnp.float32)
        pltpu.sync_copy(zb_v, hist_sh.at[pl.ds(s * _ROWS, _ROWS)])

        @pl.when(s == _NS - 1)
        def _tail_z():
            pltpu.sync_copy(zb_v.at[pl.ds(0, 16)],
                            hist_sh.at[pl.ds(_NS * _ROWS, n_nodes - _NS * _ROWS)])

        plsc.subcore_barrier()

        def chunk(i, carry):
            base = (c * _NS + s) * ept + i * _K
            pltpu.sync_copy(dst_hbm.at[pl.ds(base, _K)], idx_v)
            pltpu.sync_copy(ones_v, hist_sh.at[idx_v], add=True)
            return carry

        lax.fori_loop(0, nchunk, chunk, 0)
        plsc.subcore_barrier()
        o0 = c * n_nodes
        pltpu.sync_copy(hist_sh.at[pl.ds(s * _ROWS, _ROWS)], zb_v)
        pltpu.sync_copy(zb_v, out_hbm.at[pl.ds(o0 + s * _ROWS, _ROWS)])

        @pl.when(s == _NS - 1)
        def _tail_o():
            tail = n_nodes - _NS * _ROWS
            pltpu.sync_copy(hist_sh.at[pl.ds(_NS * _ROWS, tail)],
                            zb_v.at[pl.ds(0, tail)])
            pltpu.sync_copy(zb_v.at[pl.ds(0, tail)],
                            out_hbm.at[pl.ds(o0 + _NS * _ROWS, tail)])

    fn = pl.kernel(
        body,
        out_type=jax.ShapeDtypeStruct((_NC * n_nodes,), jnp.float32),
        mesh=_sc_mesh(),
        scratch_types=[
            pltpu.VMEM((_K,), jnp.int32),
            pltpu.VMEM((_K,), jnp.float32),
            pltpu.VMEM((_ROWS,), jnp.float32),
            pltpu.VMEM_SHARED((n_nodes,), jnp.float32),
            pltpu.SemaphoreType.DMA,
        ],
    )
    return fn(dst)


def _scatter_sc(zs, src, dst, t_steps, n_nodes, h_dim):
    e = src.shape[0]
    ept = e // _NS
    nch = ept // _K2
    tail_e = ept - nch * _K2
    tail = n_nodes - _NS * _ROWS

    def body(zs_hbm, src_hbm, dst_hbm, out_hbm,
             is0, is1, is2, is3, id0, id1, id2, id3, ra, rb,
             idxs_t, idxd_t, stg0, stg1, acc_sh,
             si0, si1, si2, si3, sg0, sg1, ss0, ss1, sw0, sw1):
        idxs_r = [is0, is1, is2, is3]
        idxd_r = [id0, id1, id2, id3]
        rows_r = [ra, rb]
        semi = [si0, si1, si2, si3]
        semg = [sg0, sg1]
        sems = [ss0, ss1]
        stg = [stg0, stg1]
        semw = [sw0, sw1]
        c = lax.axis_index("c")
        s = lax.axis_index("s")
        e0 = s * ept

        def idx_issue(i, r):
            base = e0 + i * _K2
            pltpu.async_copy(src_hbm.at[pl.ds(base, _K2)], idxs_r[r], semi[r])
            pltpu.async_copy(dst_hbm.at[pl.ds(base, _K2)], idxd_r[r], semi[r])

        def idx_wait(r):
            pltpu.make_async_copy(src_hbm.at[pl.ds(0, _K2)], idxs_r[r],
                                  semi[r]).wait()
            pltpu.make_async_copy(dst_hbm.at[pl.ds(0, _K2)], idxd_r[r],
                                  semi[r]).wait()

        def add_off(ib, n16, t_n):
            for j in range(n16):
                ib[pl.ds(j * 16, 16)] = ib[pl.ds(j * 16, 16)] + t_n

        for ti in range(t_steps // _NC):
            t = c + _NC * ti
            t_n = t * n_nodes

            def stage_io(write_out):
                npiece = _ROWS // _STG

                def piece(i, carry):
                    hb = t_n + s * _ROWS + i * _STG
                    sp = s * _ROWS + i * _STG
                    for b in range(2):
                        @pl.when(i % 2 == b)
                        def _one():
                            @pl.when(i >= 2)
                            def _drain():
                                if write_out:
                                    pltpu.make_async_copy(
                                        stg[b], out_hbm.at[pl.ds(hb, _STG)],
                                        semw[b]).wait()
                                else:
                                    pltpu.make_async_copy(
                                        stg[b], acc_sh.at[pl.ds(sp, _STG)],
                                        semw[b]).wait()
                            if write_out:
                                pltpu.sync_copy(acc_sh.at[pl.ds(sp, _STG)], stg[b])
                                pltpu.async_copy(stg[b],
                                                 out_hbm.at[pl.ds(hb, _STG)],
                                                 semw[b])
                            else:
                                pltpu.sync_copy(zs_hbm.at[pl.ds(hb, _STG)], stg[b])
                                pltpu.async_copy(stg[b],
                                                 acc_sh.at[pl.ds(sp, _STG)],
                                                 semw[b])
                    return carry
                lax.fori_loop(0, npiece, piece, 0)
                for b in range(2):
                    if write_out:
                        pltpu.make_async_copy(stg[b],
                                              out_hbm.at[pl.ds(t_n, _STG)],
                                              semw[b]).wait()
                    else:
                        pltpu.make_async_copy(stg[b],
                                              acc_sh.at[pl.ds(0, _STG)],
                                              semw[b]).wait()

                @pl.when(s == _NS - 1)
                def _tailc():
                    hb = t_n + _NS * _ROWS
                    sp = _NS * _ROWS
                    if write_out:
                        pltpu.sync_copy(acc_sh.at[pl.ds(sp, tail)],
                                        stg0.at[pl.ds(0, tail)])
                        pltpu.sync_copy(stg0.at[pl.ds(0, tail)],
                                        out_hbm.at[pl.ds(hb, tail)])
                    else:
                        pltpu.sync_copy(zs_hbm.at[pl.ds(hb, tail)],
                                        stg0.at[pl.ds(0, tail)])
                        pltpu.sync_copy(stg0.at[pl.ds(0, tail)],
                                        acc_sh.at[pl.ds(sp, tail)])

            stage_io(write_out=False)
            plsc.subcore_barrier()

            idx_issue(0, 0)
            idx_issue(1, 1)
            idx_wait(0)
            add_off(idxs_r[0], _K2 // 16, t_n)
            pltpu.async_copy(zs_hbm.at[idxs_r[0]], rows_r[0], semg[0])

            def quad(q, carry):
                for u in range(4):
                    i = 4 * q + u
                    r0, r1, r2 = u % 4, (u + 1) % 4, (u + 2) % 4
                    b, nb = u % 2, (u + 1) % 2

                    @pl.when(i > 0)
                    def _wait_prev_scatter():
                        pltpu.make_async_copy(
                            rows_r[nb], acc_sh.at[idxd_r[(u + 3) % 4]],
                            sems[nb]).wait()

                    @pl.when(i + 1 < nch)
                    def _issue_next_gather():
                        idx_wait(r1)
                        add_off(idxs_r[r1], _K2 // 16, t_n)
                        pltpu.async_copy(zs_hbm.at[idxs_r[r1]], rows_r[nb],
                                         semg[nb])

                    pltpu.make_async_copy(zs_hbm.at[idxs_r[r0]], rows_r[b],
                                          semg[b]).wait()

                    @pl.when(i + 2 < nch)
                    def _prefetch_idx():
                        idx_issue(i + 2, r2)

                    pltpu.async_copy(rows_r[b], acc_sh.at[idxd_r[r0]],
                                     sems[b], add=True)
                return carry

            lax.fori_loop(0, nch // 4, quad, 0)
            pltpu.make_async_copy(rows_r[(nch - 1) % 2],
                                  acc_sh.at[idxd_r[(nch - 1) % 4]],
                                  sems[(nch - 1) % 2]).wait()

            tb = e0 + nch * _K2
            pltpu.sync_copy(src_hbm.at[pl.ds(tb, tail_e)], idxs_t)
            pltpu.sync_copy(dst_hbm.at[pl.ds(tb, tail_e)], idxd_t)
            add_off(idxs_t, tail_e // 16, t_n)
            pltpu.async_copy(zs_hbm.at[idxs_t],
                             rows_r[0].at[pl.ds(0, _TAIL)], semg[0]).wait()
            pltpu.sync_copy(rows_r[0].at[pl.ds(0, _TAIL)],
                            acc_sh.at[idxd_t], add=True)

            plsc.subcore_barrier()
            stage_io(write_out=True)

    fn = pl.kernel(
        body,
        out_type=jax.ShapeDtypeStruct((t_steps * n_nodes, h_dim), jnp.float32),
        mesh=_sc_mesh(),
        scratch_types=(
            [pltpu.VMEM((_K2,), jnp.int32)] * 8
            + [pltpu.VMEM((_K2, h_dim), jnp.float32)] * 2
            + [pltpu.VMEM((_TAIL,), jnp.int32)] * 2
            + [pltpu.VMEM((_STG, h_dim), jnp.float32)] * 2
            + [pltpu.VMEM_SHARED((n_nodes, h_dim), jnp.float32)]
            + [pltpu.SemaphoreType.DMA] * 10
        ),
    )
    return fn(zs, src, dst)


def _zs1_tc(x, w1, deg, nb):
    t_steps, n_nodes, f_dim = x.shape
    h_dim = w1.shape[1]

    def body(x_ref, w_ref, deg_ref, o_ref):
        dinv = lax.rsqrt(deg_ref[0] + deg_ref[1] + 1.0)
        o_ref[0] = jnp.dot(x_ref[0].astype(jnp.bfloat16), w_ref[...],
                           preferred_element_type=jnp.float32) * dinv

    return pl.pallas_call(
        body,
        grid=(t_steps, n_nodes // nb),
        in_specs=[
            pl.BlockSpec((1, nb, f_dim), lambda t, i: (t, i, 0)),
            pl.BlockSpec((f_dim, h_dim), lambda t, i: (0, 0)),
            pl.BlockSpec((2, nb, 1), lambda t, i: (0, i, 0)),
        ],
        out_specs=pl.BlockSpec((1, nb, h_dim), lambda t, i: (t, i, 0)),
        out_shape=jax.ShapeDtypeStruct((t_steps, n_nodes, h_dim), jnp.float32),
    )(x, w1, deg)


def _zs2_tc(s1, w2, b1, deg, nb):
    t_steps, n_nodes, h_dim = s1.shape

    def body(s1_ref, w_ref, b_ref, deg_ref, o_ref):
        dinv = lax.rsqrt(deg_ref[0] + deg_ref[1] + 1.0)
        y = jnp.maximum(s1_ref[0] * dinv + b_ref[...], 0.0)
        o_ref[0] = jnp.dot(y.astype(jnp.bfloat16), w_ref[...],
                           preferred_element_type=jnp.float32) * dinv

    return pl.pallas_call(
        body,
        grid=(t_steps, n_nodes // nb),
        in_specs=[
            pl.BlockSpec((1, nb, h_dim), lambda t, i: (t, i, 0)),
            pl.BlockSpec((h_dim, h_dim), lambda t, i: (0, 0)),
            pl.BlockSpec((1, h_dim), lambda t, i: (0, 0)),
            pl.BlockSpec((2, nb, 1), lambda t, i: (0, i, 0)),
        ],
        out_specs=pl.BlockSpec((1, nb, h_dim), lambda t, i: (t, i, 0)),
        out_shape=jax.ShapeDtypeStruct((t_steps, n_nodes, h_dim), jnp.float32),
    )(s1, w2, b1, deg)


def _lstm_tc(s2, deg, b2, wih0t, whh0t, bs0, wih1t, whh1t, bs1, wfct, bfc, nb):
    t_steps, n_nodes, h_dim = s2.shape
    out_dim = wfct.shape[1]

    def body(s2_ref, deg_ref, b2_ref, wih0_ref, whh0_ref, bs0_ref,
             wih1_ref, whh1_ref, bs1_ref, wfc_ref, bfc_ref, o_ref):
        dinv = lax.rsqrt(deg_ref[0] + deg_ref[1] + 1.0)

        def cell(xt, h, c, wih, whh, bs):
            g = (jnp.dot(xt.astype(jnp.bfloat16), wih[...],
                         preferred_element_type=jnp.float32)
                 + jnp.dot(h.astype(jnp.bfloat16), whh[...],
                           preferred_element_type=jnp.float32)
                 + bs[...])
            i = jax.nn.sigmoid(g[:, :h_dim])
            f = jax.nn.sigmoid(g[:, h_dim:2 * h_dim])
            gg = jnp.tanh(g[:, 2 * h_dim:3 * h_dim])
            o = jax.nn.sigmoid(g[:, 3 * h_dim:])
            c = f * c + i * gg
            h = o * jnp.tanh(c)
            return h, c

        h0 = jnp.zeros((nb, h_dim), jnp.float32)
        c0 = jnp.zeros((nb, h_dim), jnp.float32)
        hs = []
        for t in range(t_steps):
            xt = jnp.maximum(s2_ref[t] * dinv + b2_ref[...], 0.0)
            h0, c0 = cell(xt, h0, c0, wih0_ref, whh0_ref, bs0_ref)
            hs.append(h0)
        h1 = jnp.zeros((nb, h_dim), jnp.float32)
        c1 = jnp.zeros((nb, h_dim), jnp.float32)
        for t in range(t_steps):
            h1, c1 = cell(hs[t], h1, c1, wih1_ref, whh1_ref, bs1_ref)
        o_ref[...] = (jnp.dot(h1.astype(jnp.bfloat16), wfc_ref[...],
                              preferred_element_type=jnp.float32)
                      + bfc_ref[...])

    full = lambda *shape: pl.BlockSpec(shape, lambda i: tuple(0 for _ in shape))
    return pl.pallas_call(
        body,
        grid=(n_nodes // nb,),
        in_specs=[
            pl.BlockSpec((t_steps, nb, h_dim), lambda i: (0, i, 0)),
            pl.BlockSpec((2, nb, 1), lambda i: (0, i, 0)),
            full(1, h_dim),
            full(h_dim, 4 * h_dim), full(h_dim, 4 * h_dim), full(1, 4 * h_dim),
            full(h_dim, 4 * h_dim), full(h_dim, 4 * h_dim), full(1, 4 * h_dim),
            full(h_dim, out_dim), full(1, out_dim),
        ],
        out_specs=pl.BlockSpec((nb, out_dim), lambda i: (i, 0)),
        out_shape=jax.ShapeDtypeStruct((n_nodes, out_dim), jnp.float32),
    )(s2, deg, b2, wih0t, whh0t, bs0, wih1t, whh1t, bs1, wfct, bfc)


def kernel(x, edge_index, W1, b1, W2, b2, Wih0, Whh0, bih0, bhh0,
           Wih1, Whh1, bih1, bhh1, Wfc, bfc):
    bsz, t_steps, n_nodes, f_dim = x.shape
    h_dim = W1.shape[1]
    src, dst = edge_index[0], edge_index[1]

    deg = _deg_sc(dst, n_nodes).reshape(2, n_nodes, 1)

    nb_mm = 2000 if n_nodes % 2000 == 0 else n_nodes
    nb_ls = 1000 if n_nodes % 1000 == 0 else n_nodes
    bf = jnp.bfloat16
    xs3 = x.reshape(t_steps, n_nodes, f_dim)
    zs1 = _zs1_tc(xs3, W1.astype(bf), deg, nb=nb_mm)
    s1 = _scatter_sc(zs1.reshape(t_steps * n_nodes, h_dim), src, dst,
                     t_steps, n_nodes, h_dim).reshape(t_steps, n_nodes, h_dim)
    zs2 = _zs2_tc(s1, W2.astype(bf), b1.reshape(1, h_dim), deg, nb=nb_mm)
    s2 = _scatter_sc(zs2.reshape(t_steps * n_nodes, h_dim), src, dst,
                     t_steps, n_nodes, h_dim).reshape(t_steps, n_nodes, h_dim)

    out = _lstm_tc(
        s2, deg, b2.reshape(1, h_dim),
        Wih0.T.astype(bf), Whh0.T.astype(bf),
        (bih0 + bhh0).reshape(1, 4 * h_dim),
        Wih1.T.astype(bf), Whh1.T.astype(bf),
        (bih1 + bhh1).reshape(1, 4 * h_dim),
        Wfc.T.astype(bf), bfc.reshape(1, -1), nb=nb_ls)
    return out.reshape(bsz, n_nodes, -1)
